# Initial kernel scaffold; baseline (speedup 1.0000x reference)
import jax
import jax.numpy as jnp
from jax.experimental import pallas as pl


def kernel(x, W):
    raise NotImplementedError("write your pallas kernel here")


# speedup vs baseline: 8.6754x; 8.6754x over previous
import jax
import jax.numpy as jnp
from jax.experimental import pallas as pl
from jax.experimental.pallas import tpu as pltpu

KNN = 20
RBLK = 256
NEG = -1e30


def _edgeconv_body(xf_ref, xb_ref, w_ref, o_ref, y1_ref, nn_ref, dist_ref):
    r = pl.program_id(1)
    xfull = xf_ref[0]
    xr = xb_ref[0]
    w1 = w_ref[:, :64]
    w2m1 = w_ref[:, 64:] - w1

    @pl.when(r == 0)
    def _():
        y1_ref[...] = jax.lax.dot(w1, xfull, preferred_element_type=jnp.float32)
        nn_ref[...] = jnp.broadcast_to(
            jnp.sum(xfull * xfull, axis=0, keepdims=True), nn_ref.shape)

    n = xfull.shape[1]
    g = jax.lax.dot_general(xr, xfull, (((0,), (0,)), ((), ())),
                            preferred_element_type=jnp.float32)
    dist_ref[...] = 2.0 * g - nn_ref[0:1, :]

    cols = jax.lax.broadcasted_iota(jnp.int32, (RBLK, n), 1)
    y1 = y1_ref[...]

    def step(_, acc):
        d = dist_ref[...]
        m = jnp.max(d, axis=1, keepdims=True)
        big = jnp.where(d == m, cols, n)
        jmin = jnp.min(big, axis=1, keepdims=True)
        onehot = cols == jmin
        dist_ref[...] = jnp.where(onehot, NEG, d)
        gsel = jax.lax.dot_general(y1, onehot.astype(jnp.float32),
                                   (((1,), (1,)), ((), ())),
                                   preferred_element_type=jnp.float32)
        return jnp.maximum(acc, gsel)

    acc = jax.lax.fori_loop(
        0, KNN, step, jnp.full((64, RBLK), NEG, jnp.float32))
    y2r = jax.lax.dot(w2m1, xr, preferred_element_type=jnp.float32)
    z = acc + y2r
    o_ref[0] = jnp.where(z >= 0, z, 0.01 * z)


def kernel(x, W):
    B, C, N = x.shape
    O = W.shape[0]
    return pl.pallas_call(
        _edgeconv_body,
        grid=(B, N // RBLK),
        in_specs=[
            pl.BlockSpec((1, C, N), lambda b, r: (b, 0, 0)),
            pl.BlockSpec((1, C, RBLK), lambda b, r: (b, 0, r)),
            pl.BlockSpec((O, 2 * C), lambda b, r: (0, 0)),
        ],
        out_specs=pl.BlockSpec((1, O, RBLK), lambda b, r: (b, 0, r)),
        out_shape=jax.ShapeDtypeStruct((B, O, N), jnp.float32),
        scratch_shapes=[
            pltpu.VMEM((O, N), jnp.float32),
            pltpu.VMEM((8, N), jnp.float32),
            pltpu.VMEM((RBLK, N), jnp.float32),
        ],
        compiler_params=pltpu.CompilerParams(
            dimension_semantics=("arbitrary", "arbitrary")),
    )(x, x, W)


# speedup vs baseline: 12.7165x; 1.4658x over previous
import functools

import jax
import jax.numpy as jnp
from jax import lax
from jax.experimental import pallas as pl
from jax.experimental.pallas import tpu as pltpu
from jax.experimental.pallas import tpu_sc as plsc

KNN = 20
RBLK = 256
NEG = -1e30

_NW = 32
_CHUNK = 32


def _tc_body(xf_ref, xb_ref, w_ref, y1t_ref, y2t_ref, idx_ref, nnt_ref,
             dist_ref):
    b = pl.program_id(0)
    r = pl.program_id(1)
    xfull = xf_ref[0]
    xr = xb_ref[0]
    n = xfull.shape[1]
    w1 = w_ref[:, :64]
    w2m1 = w_ref[:, 64:] - w1

    @pl.when(r == 0)
    def _():
        xsq = xfull * xfull
        ones = jnp.ones((xfull.shape[0], 8), jnp.float32)
        nnt_ref[...] = lax.dot_general(xsq, ones, (((0,), (0,)), ((), ())),
                                       precision=lax.Precision.HIGHEST,
                                       preferred_element_type=jnp.float32)

    w1cat = jnp.concatenate([w1, w1], axis=0)
    y1t_ref[...] = lax.dot_general(xr, w1cat, (((0,), (1,)), ((), ())),
                                   preferred_element_type=jnp.float32)
    y2t_ref[...] = lax.dot_general(xr, w2m1, (((0,), (1,)), ((), ())),
                                   preferred_element_type=jnp.float32)

    g = lax.dot_general(xfull, xr, (((0,), (0,)), ((), ())),
                        preferred_element_type=jnp.float32)
    dist_ref[...] = 2.0 * g - nnt_ref[:, 0:1]

    rows = lax.broadcasted_iota(jnp.int32, (n, RBLK), 0)
    base = b * n
    for s in range(KNN):
        d = dist_ref[...]
        m = jnp.max(d, axis=0, keepdims=True)
        big = jnp.where(d == m, rows, n)
        jmin = jnp.min(big, axis=0, keepdims=True)
        dist_ref[...] = jnp.where(rows == jmin, NEG, d)
        idx_ref[s:s + 1, :] = jmin + base


def _sc_body(y1t_hbm, idx_hbm, y2t_hbm, out_hbm, idx_v, rows_v, y2_v, out_v,
             sem):
    wid = lax.axis_index("s") * 2 + lax.axis_index("c")
    pts = y1t_hbm.shape[0] // _NW
    tile_base = wid * pts
    pltpu.sync_copy(idx_hbm.at[pl.ds(0, 24), pl.ds(tile_base, pts)], idx_v)

    def chunk(ch, _):
        base = tile_base + ch * _CHUNK
        off = ch * _CHUNK
        copies = [
            pltpu.async_copy(y1t_hbm.at[idx_v.at[s, pl.ds(off, _CHUNK)]],
                             rows_v.at[s], sem)
            for s in range(KNN)
        ]
        pltpu.sync_copy(y2t_hbm.at[pl.ds(base, _CHUNK)], y2_v)
        for c in copies:
            c.wait()

        def point(p, _):
            for c in range(4):
                sl = pl.ds(c * 16, 16)
                acc = rows_v[0, p, sl]
                for s in range(1, KNN):
                    acc = jnp.maximum(acc, rows_v[s, p, sl])
                z = acc + y2_v[p, sl]
                out_v[p, sl] = jnp.where(z >= 0, z, 0.01 * z)
            return 0

        lax.fori_loop(0, _CHUNK, point, 0)
        pltpu.sync_copy(out_v, out_hbm.at[pl.ds(base, _CHUNK)])
        return 0

    lax.fori_loop(0, pts // _CHUNK, chunk, 0)


def kernel(x, W):
    B, C, N = x.shape
    O = W.shape[0]
    nblk = N // RBLK

    y1t, y2t, idxt = pl.pallas_call(
        _tc_body,
        grid=(B, nblk),
        in_specs=[
            pl.BlockSpec((1, C, N), lambda b, r: (b, 0, 0)),
            pl.BlockSpec((1, C, RBLK), lambda b, r: (b, 0, r)),
            pl.BlockSpec((O, 2 * C), lambda b, r: (0, 0)),
        ],
        out_specs=[
            pl.BlockSpec((RBLK, 2 * C), lambda b, r: (b * (N // RBLK) + r, 0)),
            pl.BlockSpec((RBLK, C), lambda b, r: (b * (N // RBLK) + r, 0)),
            pl.BlockSpec((24, RBLK), lambda b, r: (0, b * (N // RBLK) + r)),
        ],
        out_shape=[
            jax.ShapeDtypeStruct((B * N, 2 * C), jnp.float32),
            jax.ShapeDtypeStruct((B * N, C), jnp.float32),
            jax.ShapeDtypeStruct((24, B * N), jnp.int32),
        ],
        scratch_shapes=[
            pltpu.VMEM((N, 8), jnp.float32),
            pltpu.VMEM((N, RBLK), jnp.float32),
        ],
        compiler_params=pltpu.CompilerParams(
            dimension_semantics=("arbitrary", "arbitrary")),
    )(x, x, W)

    mesh = plsc.VectorSubcoreMesh(core_axis_name="c", subcore_axis_name="s")
    outt = pl.kernel(
        _sc_body,
        mesh=mesh,
        out_type=jax.ShapeDtypeStruct((B * N, O), jnp.float32),
        scratch_types=[
            pltpu.VMEM((24, B * N // _NW), jnp.int32),
            pltpu.VMEM((KNN, _CHUNK, 2 * O), jnp.float32),
            pltpu.VMEM((_CHUNK, O), jnp.float32),
            pltpu.VMEM((_CHUNK, O), jnp.float32),
            pltpu.SemaphoreType.DMA,
        ],
    )(y1t, idxt, y2t)

    return outt.reshape(B, N, O).transpose(0, 2, 1)


# speedup vs baseline: 15.8705x; 1.2480x over previous
import functools

import jax
import jax.numpy as jnp
from jax import lax
from jax.experimental import pallas as pl
from jax.experimental.pallas import tpu as pltpu
from jax.experimental.pallas import tpu_sc as plsc

KNN = 20
RBLK = 256
NEG = -1e30

_NW = 32
_CHUNK = 32


def _tc_body(xf_ref, xb_ref, w_ref, y1t_ref, y2t_ref, idx_ref, nnt_ref,
             dist_ref):
    b = pl.program_id(0)
    r = pl.program_id(1)
    xfull = xf_ref[0]
    xr = xb_ref[0]
    n = xfull.shape[1]
    w1 = w_ref[:, :64]
    w2m1 = w_ref[:, 64:] - w1

    @pl.when(r == 0)
    def _():
        xsq = xfull * xfull
        ones = jnp.ones((xfull.shape[0], 8), jnp.float32)
        nnt_ref[...] = lax.dot_general(xsq, ones, (((0,), (0,)), ((), ())),
                                       precision=lax.Precision.HIGHEST,
                                       preferred_element_type=jnp.float32)

    w1cat = jnp.concatenate([w1, w1], axis=0)
    y1t_ref[...] = lax.dot_general(xr, w1cat, (((0,), (1,)), ((), ())),
                                   preferred_element_type=jnp.float32)
    y2t_ref[...] = lax.dot_general(xr, w2m1, (((0,), (1,)), ((), ())),
                                   preferred_element_type=jnp.float32)

    g = lax.dot_general(xfull, xr, (((0,), (0,)), ((), ())),
                        preferred_element_type=jnp.float32)
    dist_ref[...] = 2.0 * g - nnt_ref[:, 0:1]

    rows = lax.broadcasted_iota(jnp.int32, (n, RBLK), 0)
    base = b * n
    jprev = None
    for s in range(KNN):
        d = dist_ref[...]
        if jprev is not None:
            d = jnp.where(rows == jprev, NEG, d)
            dist_ref[...] = d
        jmin = jnp.argmax(d, axis=0, keepdims=True).astype(jnp.int32)
        idx_ref[s:s + 1, :] = jmin + base
        jprev = jmin


def _sc_body(y1t_hbm, idx_hbm, y2t_hbm, out_hbm, idx_v, rows_v, y2_v, out_v,
             sem):
    wid = lax.axis_index("s") * 2 + lax.axis_index("c")
    pts = y1t_hbm.shape[0] // _NW
    tile_base = wid * pts
    pltpu.sync_copy(idx_hbm.at[pl.ds(0, 24), pl.ds(tile_base, pts)], idx_v)

    def chunk(ch, _):
        base = tile_base + ch * _CHUNK
        off = ch * _CHUNK
        copies = [
            pltpu.async_copy(y1t_hbm.at[idx_v.at[s, pl.ds(off, _CHUNK)]],
                             rows_v.at[s], sem)
            for s in range(KNN)
        ]
        pltpu.sync_copy(y2t_hbm.at[pl.ds(base, _CHUNK)], y2_v)
        for c in copies:
            c.wait()

        def point(p, _):
            for c in range(4):
                sl = pl.ds(c * 16, 16)
                acc = rows_v[0, p, sl]
                for s in range(1, KNN):
                    acc = jnp.maximum(acc, rows_v[s, p, sl])
                z = acc + y2_v[p, sl]
                out_v[p, sl] = jnp.where(z >= 0, z, 0.01 * z)
            return 0

        lax.fori_loop(0, _CHUNK, point, 0)
        pltpu.sync_copy(out_v, out_hbm.at[pl.ds(base, _CHUNK)])
        return 0

    lax.fori_loop(0, pts // _CHUNK, chunk, 0)


def kernel(x, W):
    B, C, N = x.shape
    O = W.shape[0]
    nblk = N // RBLK

    y1t, y2t, idxt = pl.pallas_call(
        _tc_body,
        grid=(B, nblk),
        in_specs=[
            pl.BlockSpec((1, C, N), lambda b, r: (b, 0, 0)),
            pl.BlockSpec((1, C, RBLK), lambda b, r: (b, 0, r)),
            pl.BlockSpec((O, 2 * C), lambda b, r: (0, 0)),
        ],
        out_specs=[
            pl.BlockSpec((RBLK, 2 * C), lambda b, r: (b * (N // RBLK) + r, 0)),
            pl.BlockSpec((RBLK, C), lambda b, r: (b * (N // RBLK) + r, 0)),
            pl.BlockSpec((24, RBLK), lambda b, r: (0, b * (N // RBLK) + r)),
        ],
        out_shape=[
            jax.ShapeDtypeStruct((B * N, 2 * C), jnp.float32),
            jax.ShapeDtypeStruct((B * N, C), jnp.float32),
            jax.ShapeDtypeStruct((24, B * N), jnp.int32),
        ],
        scratch_shapes=[
            pltpu.VMEM((N, 8), jnp.float32),
            pltpu.VMEM((N, RBLK), jnp.float32),
        ],
        compiler_params=pltpu.CompilerParams(
            dimension_semantics=("arbitrary", "arbitrary")),
    )(x, x, W)

    mesh = plsc.VectorSubcoreMesh(core_axis_name="c", subcore_axis_name="s")
    outt = pl.kernel(
        _sc_body,
        mesh=mesh,
        out_type=jax.ShapeDtypeStruct((B * N, O), jnp.float32),
        scratch_types=[
            pltpu.VMEM((24, B * N // _NW), jnp.int32),
            pltpu.VMEM((KNN, _CHUNK, 2 * O), jnp.float32),
            pltpu.VMEM((_CHUNK, O), jnp.float32),
            pltpu.VMEM((_CHUNK, O), jnp.float32),
            pltpu.SemaphoreType.DMA,
        ],
    )(y1t, idxt, y2t)

    return outt.reshape(B, N, O).transpose(0, 2, 1)


# speedup vs baseline: 16.5654x; 1.0438x over previous
import functools

import jax
import jax.numpy as jnp
from jax import lax
from jax.experimental import pallas as pl
from jax.experimental.pallas import tpu as pltpu
from jax.experimental.pallas import tpu_sc as plsc

KNN = 20
RBLK = 256
NEG = -1e30

_NW = 32
_CHUNK = 32


def _tc_body(xf_ref, xb_ref, w_ref, y1t_ref, y2t_ref, idx_ref, nnt_ref,
             dist_ref):
    b = pl.program_id(0)
    r = pl.program_id(1)
    xfull = xf_ref[0]
    xr = xb_ref[0]
    n = xfull.shape[1]
    w1 = w_ref[:, :64]
    w2m1 = w_ref[:, 64:] - w1

    @pl.when(r == 0)
    def _():
        xsq = xfull * xfull
        ones = jnp.ones((xfull.shape[0], 8), jnp.float32)
        nnt_ref[...] = lax.dot_general(xsq, ones, (((0,), (0,)), ((), ())),
                                       precision=lax.Precision.HIGHEST,
                                       preferred_element_type=jnp.float32)

    w1cat = jnp.concatenate([w1, w1], axis=0)
    y1t_ref[...] = lax.dot_general(xr, w1cat, (((0,), (1,)), ((), ())),
                                   preferred_element_type=jnp.float32)
    y2t_ref[...] = lax.dot_general(xr, w2m1, (((0,), (1,)), ((), ())),
                                   preferred_element_type=jnp.float32)

    g2 = lax.dot_general(xfull, xr + xr, (((0,), (0,)), ((), ())),
                         preferred_element_type=jnp.float32)
    rows = lax.broadcasted_iota(jnp.int32, (n, RBLK), 0)
    cols = lax.broadcasted_iota(jnp.int32, (n, RBLK), 1)
    base = b * n
    selfrow = cols + r * RBLK
    dist_ref[...] = jnp.where(rows == selfrow, NEG,
                              g2 - nnt_ref[:, 0:1])
    idx_ref[0:1, :] = lax.broadcasted_iota(jnp.int32, (1, RBLK), 1) + (
        base + r * RBLK)
    jprev = None
    for s in range(1, KNN):
        d = dist_ref[...]
        if jprev is not None:
            d = jnp.where(rows == jprev, NEG, d)
            dist_ref[...] = d
        jmin = jnp.argmax(d, axis=0, keepdims=True).astype(jnp.int32)
        idx_ref[s:s + 1, :] = jmin + base
        jprev = jmin


def _sc_body(y1t_hbm, idx_hbm, y2t_hbm, out_hbm, idx_v, rows_v, y2_v, out_v,
             sem):
    wid = lax.axis_index("s") * 2 + lax.axis_index("c")
    pts = y1t_hbm.shape[0] // _NW
    tile_base = wid * pts
    pltpu.sync_copy(idx_hbm.at[pl.ds(0, 24), pl.ds(tile_base, pts)], idx_v)

    def chunk(ch, _):
        base = tile_base + ch * _CHUNK
        off = ch * _CHUNK
        copies = [
            pltpu.async_copy(y1t_hbm.at[idx_v.at[s, pl.ds(off, _CHUNK)]],
                             rows_v.at[s], sem)
            for s in range(KNN)
        ]
        pltpu.sync_copy(y2t_hbm.at[pl.ds(base, _CHUNK)], y2_v)
        for c in copies:
            c.wait()

        def point(p, _):
            for c in range(4):
                sl = pl.ds(c * 16, 16)
                acc = rows_v[0, p, sl]
                for s in range(1, KNN):
                    acc = jnp.maximum(acc, rows_v[s, p, sl])
                z = acc + y2_v[p, sl]
                out_v[p, sl] = jnp.where(z >= 0, z, 0.01 * z)
            return 0

        lax.fori_loop(0, _CHUNK, point, 0)
        pltpu.sync_copy(out_v, out_hbm.at[pl.ds(base, _CHUNK)])
        return 0

    lax.fori_loop(0, pts // _CHUNK, chunk, 0)


def kernel(x, W):
    B, C, N = x.shape
    O = W.shape[0]
    nblk = N // RBLK

    y1t, y2t, idxt = pl.pallas_call(
        _tc_body,
        grid=(B, nblk),
        in_specs=[
            pl.BlockSpec((1, C, N), lambda b, r: (b, 0, 0)),
            pl.BlockSpec((1, C, RBLK), lambda b, r: (b, 0, r)),
            pl.BlockSpec((O, 2 * C), lambda b, r: (0, 0)),
        ],
        out_specs=[
            pl.BlockSpec((RBLK, 2 * C), lambda b, r: (b * (N // RBLK) + r, 0)),
            pl.BlockSpec((RBLK, C), lambda b, r: (b * (N // RBLK) + r, 0)),
            pl.BlockSpec((24, RBLK), lambda b, r: (0, b * (N // RBLK) + r)),
        ],
        out_shape=[
            jax.ShapeDtypeStruct((B * N, 2 * C), jnp.float32),
            jax.ShapeDtypeStruct((B * N, C), jnp.float32),
            jax.ShapeDtypeStruct((24, B * N), jnp.int32),
        ],
        scratch_shapes=[
            pltpu.VMEM((N, 8), jnp.float32),
            pltpu.VMEM((N, RBLK), jnp.float32),
        ],
        compiler_params=pltpu.CompilerParams(
            dimension_semantics=("arbitrary", "arbitrary")),
    )(x, x, W)

    mesh = plsc.VectorSubcoreMesh(core_axis_name="c", subcore_axis_name="s")
    outt = pl.kernel(
        _sc_body,
        mesh=mesh,
        out_type=jax.ShapeDtypeStruct((B * N, O), jnp.float32),
        scratch_types=[
            pltpu.VMEM((24, B * N // _NW), jnp.int32),
            pltpu.VMEM((KNN, _CHUNK, 2 * O), jnp.float32),
            pltpu.VMEM((_CHUNK, O), jnp.float32),
            pltpu.VMEM((_CHUNK, O), jnp.float32),
            pltpu.SemaphoreType.DMA,
        ],
    )(y1t, idxt, y2t)

    return outt.reshape(B, N, O).transpose(0, 2, 1)


# speedup vs baseline: 17.4684x; 1.0545x over previous
import jax
import jax.numpy as jnp
from jax import lax
from jax.experimental import pallas as pl
from jax.experimental.pallas import tpu as pltpu
from jax.experimental.pallas import tpu_sc as plsc

KNN = 20
RBLK = 256
NEG = -1e30

_NW = 32
_CHUNK = 32


def _tc_body(xf_ref, xb_ref, w_ref, y1t_ref, y2t_ref, idx_ref, nnt_ref,
             dist_ref):
    b = pl.program_id(0)
    r = pl.program_id(1)
    xfull = xf_ref[0]
    xr = xb_ref[0]
    n = xfull.shape[1]
    w1 = w_ref[:, :64]
    w2m1 = w_ref[:, 64:] - w1

    @pl.when(r == 0)
    def _():
        xsq = xfull * xfull
        ones = jnp.ones((xfull.shape[0], 8), jnp.float32)
        nnt_ref[...] = lax.dot_general(xsq, ones, (((0,), (0,)), ((), ())),
                                       precision=lax.Precision.HIGHEST,
                                       preferred_element_type=jnp.float32)

    w1cat = jnp.concatenate([w1, w1], axis=0)
    y1t_ref[...] = lax.dot_general(xr, w1cat, (((0,), (1,)), ((), ())),
                                   preferred_element_type=jnp.float32)
    y2t_ref[...] = lax.dot_general(xr, w2m1, (((0,), (1,)), ((), ())),
                                   preferred_element_type=jnp.float32)

    g2 = lax.dot_general(xfull, xr + xr, (((0,), (0,)), ((), ())),
                         preferred_element_type=jnp.float32)
    rows = lax.broadcasted_iota(jnp.int32, (n, RBLK), 0)
    cols = lax.broadcasted_iota(jnp.int32, (n, RBLK), 1)
    base = b * n
    selfrow = cols + r * RBLK
    dist_ref[...] = jnp.where(rows == selfrow, NEG,
                              g2 - nnt_ref[:, 0:1])
    idx_ref[0:1, :] = lax.broadcasted_iota(jnp.int32, (1, RBLK), 1) + (
        base + r * RBLK)
    jprev = None
    for s in range(1, KNN):
        d = dist_ref[...]
        if jprev is not None:
            d = jnp.where(rows == jprev, NEG, d)
            dist_ref[...] = d
        jmin = jnp.argmax(d, axis=0, keepdims=True).astype(jnp.int32)
        idx_ref[s:s + 1, :] = jmin + base
        jprev = jmin


def _sc_body(y1t_hbm, idx_hbm, y2t_hbm, out_hbm, idx_v, rows_v, y2_v, out_v,
             sem):
    wid = lax.axis_index("s") * 2 + lax.axis_index("c")
    pts = y1t_hbm.shape[0] // _NW
    tile_base = wid * pts
    pltpu.sync_copy(idx_hbm.at[pl.ds(0, 24), pl.ds(tile_base, pts)], idx_v)

    def chunk(ch, _):
        base = tile_base + ch * _CHUNK
        off = ch * _CHUNK
        copies = [
            pltpu.async_copy(y1t_hbm.at[idx_v.at[s, pl.ds(off, _CHUNK)]],
                             rows_v.at[s], sem)
            for s in range(KNN)
        ]
        pltpu.sync_copy(y2t_hbm.at[pl.ds(base, _CHUNK)], y2_v)
        for c in copies:
            c.wait()

        def point(p, _):
            for c in range(4):
                sl = pl.ds(c * 16, 16)
                acc = rows_v[0, p, sl]
                for s in range(1, KNN):
                    acc = jnp.maximum(acc, rows_v[s, p, sl])
                z = acc + y2_v[p, sl]
                out_v[p, sl] = jnp.where(z >= 0, z, 0.01 * z)
            return 0

        lax.fori_loop(0, _CHUNK, point, 0)
        pltpu.sync_copy(out_v, out_hbm.at[pl.ds(base, _CHUNK)])
        return 0

    lax.fori_loop(0, pts // _CHUNK, chunk, 0)


def _half(xh, W):
    B, C, N = xh.shape
    O = W.shape[0]
    nblk = N // RBLK

    y1t, y2t, idxt = pl.pallas_call(
        _tc_body,
        grid=(B, nblk),
        in_specs=[
            pl.BlockSpec((1, C, N), lambda b, r: (b, 0, 0)),
            pl.BlockSpec((1, C, RBLK), lambda b, r: (b, 0, r)),
            pl.BlockSpec((O, 2 * C), lambda b, r: (0, 0)),
        ],
        out_specs=[
            pl.BlockSpec((RBLK, 2 * C), lambda b, r: (b * (N // RBLK) + r, 0)),
            pl.BlockSpec((RBLK, C), lambda b, r: (b * (N // RBLK) + r, 0)),
            pl.BlockSpec((24, RBLK), lambda b, r: (0, b * (N // RBLK) + r)),
        ],
        out_shape=[
            jax.ShapeDtypeStruct((B * N, 2 * C), jnp.float32),
            jax.ShapeDtypeStruct((B * N, C), jnp.float32),
            jax.ShapeDtypeStruct((24, B * N), jnp.int32),
        ],
        scratch_shapes=[
            pltpu.VMEM((N, 8), jnp.float32),
            pltpu.VMEM((N, RBLK), jnp.float32),
        ],
        compiler_params=pltpu.CompilerParams(
            dimension_semantics=("arbitrary", "arbitrary")),
    )(xh, xh, W)

    mesh = plsc.VectorSubcoreMesh(core_axis_name="c", subcore_axis_name="s")
    outt = pl.kernel(
        _sc_body,
        mesh=mesh,
        out_type=jax.ShapeDtypeStruct((B * N, O), jnp.float32),
        scratch_types=[
            pltpu.VMEM((24, B * N // _NW), jnp.int32),
            pltpu.VMEM((KNN, _CHUNK, 2 * O), jnp.float32),
            pltpu.VMEM((_CHUNK, O), jnp.float32),
            pltpu.VMEM((_CHUNK, O), jnp.float32),
            pltpu.SemaphoreType.DMA,
        ],
    )(y1t, idxt, y2t)

    return outt.reshape(B, N, O)


def kernel(x, W):
    B, C, N = x.shape
    h = B // 2
    out0 = _half(x[:h], W)
    out1 = _half(x[h:], W)
    return jnp.concatenate([out0, out1], axis=0).transpose(0, 2, 1)


# speedup vs baseline: 19.8887x; 1.1386x over previous
import jax
import jax.numpy as jnp
from jax import lax
from jax.experimental import pallas as pl
from jax.experimental.pallas import tpu as pltpu
from jax.experimental.pallas import tpu_sc as plsc

KNN = 20
RBLK = 256
NEG = -1e30

_NW = 32
_CHUNK = 32


def _tc_body(xf_ref, xb_ref, w_ref, y1t_ref, y2t_ref, idx_ref, nnt_ref,
             dist_ref):
    b = pl.program_id(0)
    r = pl.program_id(1)
    xfull = xf_ref[0]
    xr = xb_ref[0]
    n = xfull.shape[1]
    w1 = w_ref[:, :64]
    w2m1 = w_ref[:, 64:] - w1

    @pl.when(r == 0)
    def _():
        xsq = xfull * xfull
        ones = jnp.ones((xfull.shape[0], 8), jnp.float32)
        nnt_ref[...] = lax.dot_general(xsq, ones, (((0,), (0,)), ((), ())),
                                       precision=lax.Precision.HIGHEST,
                                       preferred_element_type=jnp.float32)

    w1cat = jnp.concatenate([w1, w1], axis=0)
    y1t_ref[...] = lax.dot_general(xr, w1cat, (((0,), (1,)), ((), ())),
                                   preferred_element_type=jnp.float32)
    y2t_ref[...] = lax.dot_general(xr, w2m1, (((0,), (1,)), ((), ())),
                                   preferred_element_type=jnp.float32)

    rows = lax.broadcasted_iota(jnp.int32, (n, RBLK), 0)
    cols = lax.broadcasted_iota(jnp.int32, (n, RBLK), 1)
    nblk = n // RBLK
    base = b * n

    def stage(dst, xcols, blk):
        g2 = lax.dot_general(xfull, xcols + xcols, (((0,), (0,)), ((), ())),
                             preferred_element_type=jnp.float32)
        dist_ref[dst] = jnp.where(rows == cols + blk * RBLK, NEG,
                                  g2 - nnt_ref[:, 0:1])

    @pl.when(r == 0)
    def _():
        stage(0, xr, r)

    @pl.when(r + 1 < nblk)
    def _():
        xnext = xf_ref[0, :, pl.ds(pl.multiple_of((r + 1) * RBLK, RBLK), RBLK)]
        stage((r + 1) % 2, xnext, r + 1)

    r2 = r % 2
    idx_ref[0:1, :] = lax.broadcasted_iota(jnp.int32, (1, RBLK), 1) + (
        base + r * RBLK)
    jprev = None
    for s in range(1, KNN):
        d = dist_ref[r2]
        if jprev is not None:
            d = jnp.where(rows == jprev, NEG, d)
            dist_ref[r2] = d
        jmin = jnp.argmax(d, axis=0, keepdims=True).astype(jnp.int32)
        idx_ref[s:s + 1, :] = jmin + base
        jprev = jmin


def _sc_body(y1t_hbm, idx_hbm, y2t_hbm, out_hbm, idx_v, rows_v, y2_v, out_v,
             sem):
    wid = lax.axis_index("s") * 2 + lax.axis_index("c")
    pts = y1t_hbm.shape[0] // _NW
    tile_base = wid * pts
    pltpu.sync_copy(idx_hbm.at[pl.ds(0, 24), pl.ds(tile_base, pts)], idx_v)

    def chunk(ch, _):
        base = tile_base + ch * _CHUNK
        off = ch * _CHUNK
        copies = [
            pltpu.async_copy(y1t_hbm.at[idx_v.at[s, pl.ds(off, _CHUNK)]],
                             rows_v.at[s], sem)
            for s in range(KNN)
        ]
        pltpu.sync_copy(y2t_hbm.at[pl.ds(base, _CHUNK)], y2_v)
        for c in copies:
            c.wait()

        def point(p, _):
            for c in range(4):
                sl = pl.ds(c * 16, 16)
                acc = rows_v[0, p, sl]
                for s in range(1, KNN):
                    acc = jnp.maximum(acc, rows_v[s, p, sl])
                z = acc + y2_v[p, sl]
                out_v[p, sl] = jnp.where(z >= 0, z, 0.01 * z)
            return 0

        lax.fori_loop(0, _CHUNK, point, 0)
        pltpu.sync_copy(out_v, out_hbm.at[pl.ds(base, _CHUNK)])
        return 0

    lax.fori_loop(0, pts // _CHUNK, chunk, 0)


def _half(xh, W):
    B, C, N = xh.shape
    O = W.shape[0]
    nblk = N // RBLK

    y1t, y2t, idxt = pl.pallas_call(
        _tc_body,
        grid=(B, nblk),
        in_specs=[
            pl.BlockSpec((1, C, N), lambda b, r: (b, 0, 0)),
            pl.BlockSpec((1, C, RBLK), lambda b, r: (b, 0, r)),
            pl.BlockSpec((O, 2 * C), lambda b, r: (0, 0)),
        ],
        out_specs=[
            pl.BlockSpec((RBLK, 2 * C), lambda b, r: (b * (N // RBLK) + r, 0)),
            pl.BlockSpec((RBLK, C), lambda b, r: (b * (N // RBLK) + r, 0)),
            pl.BlockSpec((24, RBLK), lambda b, r: (0, b * (N // RBLK) + r)),
        ],
        out_shape=[
            jax.ShapeDtypeStruct((B * N, 2 * C), jnp.float32),
            jax.ShapeDtypeStruct((B * N, C), jnp.float32),
            jax.ShapeDtypeStruct((24, B * N), jnp.int32),
        ],
        scratch_shapes=[
            pltpu.VMEM((N, 8), jnp.float32),
            pltpu.VMEM((2, N, RBLK), jnp.float32),
        ],
        compiler_params=pltpu.CompilerParams(
            dimension_semantics=("arbitrary", "arbitrary")),
    )(xh, xh, W)

    mesh = plsc.VectorSubcoreMesh(core_axis_name="c", subcore_axis_name="s")
    outt = pl.kernel(
        _sc_body,
        mesh=mesh,
        out_type=jax.ShapeDtypeStruct((B * N, O), jnp.float32),
        scratch_types=[
            pltpu.VMEM((24, B * N // _NW), jnp.int32),
            pltpu.VMEM((KNN, _CHUNK, 2 * O), jnp.float32),
            pltpu.VMEM((_CHUNK, O), jnp.float32),
            pltpu.VMEM((_CHUNK, O), jnp.float32),
            pltpu.SemaphoreType.DMA,
        ],
    )(y1t, idxt, y2t)

    return outt.reshape(B, N, O)


def kernel(x, W):
    B, C, N = x.shape
    h = B // 2
    out0 = _half(x[:h], W)
    out1 = _half(x[h:], W)
    return jnp.concatenate([out0, out1], axis=0).transpose(0, 2, 1)


# speedup vs baseline: 20.6146x; 1.0365x over previous
import jax
import jax.numpy as jnp
from jax import lax
from jax.experimental import pallas as pl
from jax.experimental.pallas import tpu as pltpu
from jax.experimental.pallas import tpu_sc as plsc

KNN = 20
RBLK = 256
NEG = -1e30

_NW = 32
_CHUNK = 32


def _tc_body(xf_ref, xb_ref, w_ref, y1t_ref, y2t_ref, idx_ref, nnt_ref,
             dist_ref):
    b = pl.program_id(0)
    r = pl.program_id(1)
    xfull = xf_ref[0]
    xr = xb_ref[0]
    n = xfull.shape[1]
    w1 = w_ref[:, :64]
    w2m1 = w_ref[:, 64:] - w1

    @pl.when(r == 0)
    def _():
        xsq = xfull * xfull
        ones = jnp.ones((xfull.shape[0], 8), jnp.float32)
        nnt_ref[...] = lax.dot_general(xsq, ones, (((0,), (0,)), ((), ())),
                                       precision=lax.Precision.HIGHEST,
                                       preferred_element_type=jnp.float32)

    w1cat = jnp.concatenate([w1, w1], axis=0)
    y1t_ref[...] = lax.dot_general(xr, w1cat, (((0,), (1,)), ((), ())),
                                   preferred_element_type=jnp.float32)
    y2t_ref[...] = lax.dot_general(xr, w2m1, (((0,), (1,)), ((), ())),
                                   preferred_element_type=jnp.float32)

    rows = lax.broadcasted_iota(jnp.int32, (n, RBLK), 0)
    cols = lax.broadcasted_iota(jnp.int32, (n, RBLK), 1)
    nblk = n // RBLK
    base = b * n

    def stage(dst, xcols, blk):
        g2 = lax.dot_general(xfull, xcols + xcols, (((0,), (0,)), ((), ())),
                             preferred_element_type=jnp.float32)
        dist_ref[dst] = jnp.where(rows == cols + blk * RBLK, NEG,
                                  g2 - nnt_ref[:, 0:1])

    @pl.when(r == 0)
    def _():
        stage(0, xr, r)

    @pl.when(r + 1 < nblk)
    def _():
        xnext = xf_ref[0, :, pl.ds(pl.multiple_of((r + 1) * RBLK, RBLK), RBLK)]
        stage((r + 1) % 2, xnext, r + 1)

    r2 = r % 2
    idx_ref[0:1, :] = lax.broadcasted_iota(jnp.int32, (1, RBLK), 1) + (
        base + r * RBLK)
    jprev = None
    for s in range(1, KNN):
        d = dist_ref[r2]
        if jprev is not None:
            d = jnp.where(rows == jprev, NEG, d)
            dist_ref[r2] = d
        jmin = jnp.argmax(d, axis=0, keepdims=True).astype(jnp.int32)
        idx_ref[s:s + 1, :] = jmin + base
        jprev = jmin


def _sc_body(y1t_hbm, idx_hbm, y2t_hbm, out_hbm, idx_v, rows_v, y2_v, out_v,
             sem):
    wid = lax.axis_index("s") * 2 + lax.axis_index("c")
    pts = y1t_hbm.shape[0] // _NW
    tile_base = wid * pts
    pltpu.sync_copy(idx_hbm.at[pl.ds(0, 24), pl.ds(tile_base, pts)], idx_v)

    def chunk(ch, _):
        base = tile_base + ch * _CHUNK
        off = ch * _CHUNK
        copies = [
            pltpu.async_copy(y1t_hbm.at[idx_v.at[s, pl.ds(off, _CHUNK)]],
                             rows_v.at[s], sem)
            for s in range(KNN)
        ]
        pltpu.sync_copy(y2t_hbm.at[pl.ds(base, _CHUNK)], y2_v)
        for c in copies:
            c.wait()

        def point(p, _):
            for c in range(4):
                sl = pl.ds(c * 16, 16)
                acc = rows_v[0, p, sl]
                for s in range(1, KNN):
                    acc = jnp.maximum(acc, rows_v[s, p, sl])
                z = acc + y2_v[p, sl]
                out_v[p, sl] = jnp.where(z >= 0, z, 0.01 * z)
            return 0

        lax.fori_loop(0, _CHUNK, point, 0)
        pltpu.sync_copy(out_v, out_hbm.at[pl.ds(base, _CHUNK)])
        return 0

    lax.fori_loop(0, pts // _CHUNK, chunk, 0)


def _half(xh, W):
    B, C, N = xh.shape
    O = W.shape[0]
    nblk = N // RBLK

    y1t, y2t, idxt = pl.pallas_call(
        _tc_body,
        grid=(B, nblk),
        in_specs=[
            pl.BlockSpec((1, C, N), lambda b, r: (b, 0, 0)),
            pl.BlockSpec((1, C, RBLK), lambda b, r: (b, 0, r)),
            pl.BlockSpec((O, 2 * C), lambda b, r: (0, 0)),
        ],
        out_specs=[
            pl.BlockSpec((RBLK, 2 * C), lambda b, r: (b * (N // RBLK) + r, 0)),
            pl.BlockSpec((RBLK, C), lambda b, r: (b * (N // RBLK) + r, 0)),
            pl.BlockSpec((24, RBLK), lambda b, r: (0, b * (N // RBLK) + r)),
        ],
        out_shape=[
            jax.ShapeDtypeStruct((B * N, 2 * C), jnp.float32),
            jax.ShapeDtypeStruct((B * N, C), jnp.float32),
            jax.ShapeDtypeStruct((24, B * N), jnp.int32),
        ],
        scratch_shapes=[
            pltpu.VMEM((N, 8), jnp.float32),
            pltpu.VMEM((2, N, RBLK), jnp.float32),
        ],
        compiler_params=pltpu.CompilerParams(
            dimension_semantics=("arbitrary", "arbitrary")),
    )(xh, xh, W)

    mesh = plsc.VectorSubcoreMesh(core_axis_name="c", subcore_axis_name="s")
    outt = pl.kernel(
        _sc_body,
        mesh=mesh,
        out_type=jax.ShapeDtypeStruct((B * N, O), jnp.float32),
        scratch_types=[
            pltpu.VMEM((24, B * N // _NW), jnp.int32),
            pltpu.VMEM((KNN, _CHUNK, 2 * O), jnp.float32),
            pltpu.VMEM((_CHUNK, O), jnp.float32),
            pltpu.VMEM((_CHUNK, O), jnp.float32),
            pltpu.SemaphoreType.DMA,
        ],
    )(y1t, idxt, y2t)

    return outt.reshape(B, N, O)


def kernel(x, W):
    B, C, N = x.shape
    h = B // 4
    outs = [_half(x[i * h:(i + 1) * h], W) for i in range(4)]
    return jnp.concatenate(outs, axis=0).transpose(0, 2, 1)
